# baseline (speedup 1.0000x reference)
import functools

import jax
import jax.numpy as jnp
from jax import lax
from jax.experimental import pallas as pl
from jax.experimental.pallas import tpu as pltpu
from jax.experimental.pallas import tpu_sc as plsc

NUM_FIELDS = 26
VOCAB = 100000
EMB_DIM = 32
BATCH = 16384
NUM_DIM = 13
EPS = 1e-5

NC = 2
NS = 16
NW = NC * NS

ROWS_PER_W = BATCH // NW
NB = 64
CHUNKS = ROWS_PER_W // NB
IDX_PER_CHUNK = NB * NUM_FIELDS
GATHER_LEN = 128
GATHERS = IDX_PER_CHUNK // GATHER_LEN


def _prep_body(cat_ref, num_ref, gamma_ref, beta_ref, fidx_ref, numout_ref):
    idx = jnp.clip(cat_ref[...], 0, VOCAB - 1)
    field_off = jax.lax.broadcasted_iota(jnp.int32, (1, NUM_FIELDS), 1) * VOCAB
    fidx_ref[...] = idx + field_off
    x = num_ref[...]
    mean = jnp.mean(x, axis=0, keepdims=True)
    var = jnp.mean((x - mean) * (x - mean), axis=0, keepdims=True)
    numout_ref[...] = (x - mean) * jax.lax.rsqrt(var + EPS) * gamma_ref[...] \
        + beta_ref[...]


def _prep(categorical_inputs, numerical_inputs, bn_gamma, bn_beta):
    return pl.pallas_call(
        _prep_body,
        out_shape=(
            jax.ShapeDtypeStruct((BATCH, NUM_FIELDS), jnp.int32),
            jax.ShapeDtypeStruct((BATCH, NUM_DIM), jnp.float32),
        ),
    )(categorical_inputs, numerical_inputs,
      bn_gamma.reshape(1, NUM_DIM), bn_beta.reshape(1, NUM_DIM))


def _gather_body(fidx_hbm, table_hbm, out_hbm, idx_v, rows_v, gsem):
    wid = lax.axis_index("s") * NC + lax.axis_index("c")
    for c in range(CHUNKS):
        chunk = wid * CHUNKS + c
        base = chunk * IDX_PER_CHUNK
        pltpu.sync_copy(fidx_hbm.at[pl.ds(base, IDX_PER_CHUNK)], idx_v)
        handles = []
        for j in range(GATHERS):
            handles.append(pltpu.async_copy(
                table_hbm.at[idx_v.at[pl.ds(j * GATHER_LEN, GATHER_LEN)]],
                rows_v.at[pl.ds(j * GATHER_LEN, GATHER_LEN), :],
                gsem))
        for h in handles:
            h.wait()
        pltpu.sync_copy(rows_v, out_hbm.at[pl.ds(base, IDX_PER_CHUNK), :])


@functools.partial(
    pl.kernel,
    mesh=plsc.VectorSubcoreMesh(core_axis_name="c", subcore_axis_name="s"),
    out_type=jax.ShapeDtypeStruct((BATCH * NUM_FIELDS, EMB_DIM), jnp.float32),
    compiler_params=pltpu.CompilerParams(use_tc_tiling_on_sc=False),
    scratch_types=[
        pltpu.VMEM((IDX_PER_CHUNK,), jnp.int32),
        pltpu.VMEM((IDX_PER_CHUNK, EMB_DIM), jnp.float32),
        pltpu.SemaphoreType.DMA,
    ],
)
def _gather(fidx_hbm, table_hbm, out_hbm, idx_v, rows_v, gsem):
    _gather_body(fidx_hbm, table_hbm, out_hbm, idx_v, rows_v, gsem)


def kernel(categorical_inputs, numerical_inputs, tables, bn_gamma, bn_beta):
    fidx, num_out = _prep(categorical_inputs, numerical_inputs,
                          bn_gamma, bn_beta)
    flat_table = tables.reshape(NUM_FIELDS * VOCAB, EMB_DIM)
    cat = _gather(fidx.reshape(BATCH * NUM_FIELDS), flat_table)
    cat = cat.reshape(BATCH, NUM_FIELDS * EMB_DIM)
    return jnp.concatenate([cat, num_out], axis=1)
